# baseline (speedup 1.0000x reference)
import functools

import jax
import jax.numpy as jnp
from jax import lax
from jax.experimental import pallas as pl
from jax.experimental.pallas import tpu as pltpu
from jax.experimental.pallas import tpu_sc as plsc

N_NODES = 10000
N_EDGES = 320000
D = 128
DEG = 32
B = 4096

NC = 2
NS = 16
NW = NC * NS
E_PER_W = B // NW
N_PER_W = 2 * E_PER_W
L = 16
VPD = D // L
NBUF = 2
NPS = 4
STEPS = N_PER_W // NPS

_ROWS_BLK = 400


def _tc_body(f_ref, w1_ref, w2_ref, h1_ref, h2_ref):
    f = f_ref[...]
    dn = (((1,), (1,)), ((), ()))
    h1_ref[...] = lax.dot_general(f, w1_ref[...], dn,
                                  preferred_element_type=jnp.float32)
    h2 = lax.dot_general(f, w2_ref[...], dn,
                         preferred_element_type=jnp.float32)
    h2_ref[...] = h2 * (1.0 / DEG)


def _tc_encode(features, w1, w2):
    grid = (N_NODES // _ROWS_BLK,)
    return pl.pallas_call(
        _tc_body,
        grid=grid,
        in_specs=[
            pl.BlockSpec((_ROWS_BLK, D), lambda i: (i, 0)),
            pl.BlockSpec((D, D), lambda i: (0, 0)),
            pl.BlockSpec((D, D), lambda i: (0, 0)),
        ],
        out_specs=[
            pl.BlockSpec((_ROWS_BLK, D), lambda i: (i, 0)),
            pl.BlockSpec((_ROWS_BLK, D), lambda i: (i, 0)),
        ],
        out_shape=[
            jax.ShapeDtypeStruct((N_NODES, D), jnp.float32),
            jax.ShapeDtypeStruct((N_NODES, D), jnp.float32),
        ],
    )(features, w1, w2)


def _sc_body(node_hbm, nidx_hbm, h1_hbm, h2_hbm, w_hbm, out_hbm,
             node_v, neigh_v, sfeat_v, bufs, wv_v, sc_v, out_v,
             sem_s, sem0, sem1):
    sems = (sem0, sem1)
    wid = lax.axis_index("s") * NC + lax.axis_index("c")
    base_e = wid * E_PER_W
    base_n = wid * N_PER_W

    pltpu.sync_copy(node_hbm.at[pl.ds(base_n, N_PER_W)], node_v)
    pltpu.sync_copy(nidx_hbm.at[pl.ds(base_n * DEG, N_PER_W * DEG)], neigh_v)
    cp_s = pltpu.async_copy(h1_hbm.at[node_v], sfeat_v, sem_s)
    pltpu.sync_copy(w_hbm, wv_v)

    for b in range(NBUF):
        pltpu.async_copy(h2_hbm.at[neigh_v.at[pl.ds(b * NPS * DEG, NPS * DEG)]],
                         bufs.at[b], sems[b])
    cp_s.wait()

    w_regs = [wv_v[pl.ds(v * L, L)] for v in range(VPD)]

    def _chunk(i, _):
        for b in range(NBUF):
            st = i * NBUF + b
            pltpu.make_async_copy(
                h2_hbm.at[neigh_v.at[pl.ds(st * NPS * DEG, NPS * DEG)]],
                bufs.at[b], sems[b]).wait()
            for q in range(NPS):
                n = st * NPS + q
                acc = [bufs[b, q * DEG, pl.ds(v * L, L)] for v in range(VPD)]
                for j in range(1, DEG):
                    for v in range(VPD):
                        acc[v] = acc[v] + bufs[b, q * DEG + j, pl.ds(v * L, L)]
                r = None
                for v in range(VPD):
                    z = jnp.maximum(acc[v] + sfeat_v[n, pl.ds(v * L, L)], 0.0)
                    t = z * w_regs[v]
                    r = t if r is None else r + t
                sc_v[n, pl.ds(0, L)] = r
            nst = st + NBUF

            @pl.when(nst < STEPS)
            def _issue():
                pltpu.async_copy(
                    h2_hbm.at[neigh_v.at[pl.ds(nst * NPS * DEG, NPS * DEG)]],
                    bufs.at[b], sems[b])
        return _

    lax.fori_loop(0, STEPS // NBUF, _chunk, None)

    for blk in range(E_PER_W // L):
        p = lax.iota(jnp.int32, L) + jnp.int32(blk * L)
        acc = None
        for c in range(L):
            cv = jnp.full((L,), c, jnp.int32)
            t = (plsc.load_gather(sc_v, [p * 2, cv])
                 + plsc.load_gather(sc_v, [p * 2 + 1, cv]))
            acc = t if acc is None else acc + t
        out_v[pl.ds(blk * L, L)] = acc * 0.5

    pltpu.sync_copy(out_v, out_hbm.at[pl.ds(base_e, E_PER_W)])


@functools.cache
def _sc_gather_fn():
  return pl.kernel(
    _sc_body,
    out_type=jax.ShapeDtypeStruct((B,), jnp.float32),
    mesh=plsc.VectorSubcoreMesh(core_axis_name="c", subcore_axis_name="s",
                                num_cores=NC, num_subcores=NS),
    compiler_params=pltpu.CompilerParams(needs_layout_passes=False),
    scratch_types=[
        pltpu.VMEM((N_PER_W,), jnp.int32),
        pltpu.VMEM((N_PER_W * DEG,), jnp.int32),
        pltpu.VMEM((N_PER_W, D), jnp.float32),
        pltpu.VMEM((NBUF, NPS * DEG, D), jnp.float32),
        pltpu.VMEM((D,), jnp.float32),
        pltpu.VMEM((N_PER_W, L), jnp.float32),
        pltpu.VMEM((E_PER_W,), jnp.float32),
        pltpu.SemaphoreType.DMA,
        pltpu.SemaphoreType.DMA,
        pltpu.SemaphoreType.DMA,
    ],
  )


def kernel(edges, edge_pairs, neigh, features, W_enc, weight):
    w1 = W_enc[:, :D]
    w2 = W_enc[:, D:]
    h1, h2 = _tc_encode(features, w1, w2)
    node_list = jnp.take(edge_pairs, edges, axis=0).reshape(-1)
    neigh_flat = jnp.take(neigh, node_list, axis=0).reshape(-1)
    scores = _sc_gather_fn()(node_list, neigh_flat, h1, h2, weight.reshape(D))
    return scores.reshape(B, 1)


# speedup vs baseline: 1.6506x; 1.6506x over previous
import functools

import jax
import jax.numpy as jnp
import numpy as np
from jax import lax
from jax.experimental import pallas as pl
from jax.experimental.pallas import tpu as pltpu
from jax.experimental.pallas import tpu_sc as plsc

N_NODES = 10000
N_EDGES = 320000
D = 128
DEG = 32
B = 4096

NC = 2
NS = 16
NW = NC * NS
E_PER_W = B // NW
N_PER_W = 2 * E_PER_W
L = 16
VPD = D // L
NBUF = 2
NPS = 1
STEPS = N_PER_W // NPS

_ROWS_BLK = 400


def _tc_body(f_ref, w1_ref, w2_ref, h1_ref, h2_ref):
    f = f_ref[...]
    dn = (((1,), (1,)), ((), ()))
    h1_ref[...] = lax.dot_general(f, w1_ref[...], dn,
                                  preferred_element_type=jnp.float32)
    h2 = lax.dot_general(f, w2_ref[...], dn,
                         preferred_element_type=jnp.float32)
    h2_ref[...] = h2 * (1.0 / DEG)


def _tc_encode(features, w1, w2):
    grid = (N_NODES // _ROWS_BLK,)
    return pl.pallas_call(
        _tc_body,
        grid=grid,
        in_specs=[
            pl.BlockSpec((_ROWS_BLK, D), lambda i: (i, 0)),
            pl.BlockSpec((D, D), lambda i: (0, 0)),
            pl.BlockSpec((D, D), lambda i: (0, 0)),
        ],
        out_specs=[
            pl.BlockSpec((_ROWS_BLK, D), lambda i: (i, 0)),
            pl.BlockSpec((_ROWS_BLK, D), lambda i: (i, 0)),
        ],
        out_shape=[
            jax.ShapeDtypeStruct((N_NODES, D), jnp.float32),
            jax.ShapeDtypeStruct((N_NODES, D), jnp.float32),
        ],
    )(features, w1, w2)


def _sc_body(node_hbm, nidx_hbm, h1_hbm, h2_hbm, w_hbm, out_hbm,
             node_v, neigh_v, sfb, bufs, wv_v, sc_v, out_v, h2_sh,
             semf0, semf1, sem0, sem1):
    sems = (sem0, sem1)
    semf = (semf0, semf1)
    sid = lax.axis_index("s")
    wid = sid * NC + lax.axis_index("c")
    base_e = wid * E_PER_W
    base_n = wid * N_PER_W
    NCH = N_PER_W // 32

    pltpu.sync_copy(node_hbm.at[pl.ds(base_n, N_PER_W)], node_v)
    pltpu.sync_copy(nidx_hbm.at[pl.ds(base_n * DEG, N_PER_W * DEG)], neigh_v)
    pltpu.sync_copy(w_hbm, wv_v)

    stage_base = pl.multiple_of(sid * 624, 8)
    pltpu.sync_copy(h2_hbm.at[pl.ds(stage_base, 624)],
                    h2_sh.at[pl.ds(stage_base, 624)])

    @pl.when(sid == 0)
    def _tail():
        pltpu.sync_copy(h2_hbm.at[pl.ds(624 * NS, N_NODES - 624 * NS)],
                        h2_sh.at[pl.ds(624 * NS, N_NODES - 624 * NS)])
    plsc.subcore_barrier()

    for cc in range(2):
        pltpu.async_copy(h1_hbm.at[node_v.at[pl.ds(cc * 32, 32)]],
                         sfb.at[cc], semf[cc])
    for b in range(NBUF):
        pltpu.async_copy(h2_sh.at[neigh_v.at[pl.ds(b * DEG, DEG)]],
                         bufs.at[b], sems[b])

    w_regs = [wv_v[pl.ds(v * L, L)] for v in range(VPD)]

    def _outer(i, _):
        for cc in range(2):
            c = i * 2 + cc
            pltpu.make_async_copy(h1_hbm.at[node_v.at[pl.ds(c * 32, 32)]],
                                  sfb.at[cc], semf[cc]).wait()

            def _inner(k, _2):
                rr = [None, None]
                for b in range(NBUF):
                    nl = k * NBUF + b
                    n = c * 32 + nl
                    pltpu.make_async_copy(
                        h2_sh.at[neigh_v.at[pl.ds(n * DEG, DEG)]],
                        bufs.at[b], sems[b]).wait()
                    def _accum(jj, a):
                        out = list(a)
                        for dj in range(4):
                            for v in range(VPD):
                                out[v] = out[v] + bufs[b, jj * 4 + dj,
                                                       pl.ds(v * L, L)]
                        return tuple(out)

                    acc = lax.fori_loop(
                        0, DEG // 4, _accum,
                        tuple(jnp.zeros((L,), jnp.float32)
                              for _ in range(VPD)))
                    nn = n + NBUF

                    @pl.when(nn < N_PER_W)
                    def _issue():
                        pltpu.async_copy(
                            h2_sh.at[neigh_v.at[pl.ds(nn * DEG, DEG)]],
                            bufs.at[b], sems[b])

                    r = None
                    for v in range(VPD):
                        z = jnp.maximum(acc[v] + sfb[cc, nl, pl.ds(v * L, L)],
                                        0.0)
                        t = z * w_regs[v]
                        r = t if r is None else r + t
                    e_half = b & 1
                    if e_half == 0:
                        rr[b // 2] = r
                    else:
                        edge = c * 16 + k * (NBUF // 2) + b // 2
                        sc_v[edge, pl.ds(0, L)] = rr[b // 2] + r
                return _2

            lax.fori_loop(0, 32 // NBUF, _inner, None)

            nc2 = c + 2

            @pl.when(nc2 < NCH)
            def _issue_sf():
                pltpu.async_copy(h1_hbm.at[node_v.at[pl.ds(nc2 * 32, 32)]],
                                 sfb.at[cc], semf[cc])
        return _

    lax.fori_loop(0, NCH // 2, _outer, None)

    for blk in range(E_PER_W // L):
        p = lax.iota(jnp.int32, L) + jnp.int32(blk * L)
        acc = None
        for cidx in range(L):
            cv = jnp.full((L,), cidx, jnp.int32)
            t = plsc.load_gather(sc_v, [p, cv])
            acc = t if acc is None else acc + t
        out_v[pl.ds(blk * L, L)] = acc * 0.5

    pltpu.sync_copy(out_v, out_hbm.at[pl.ds(base_e, E_PER_W)])


@functools.cache
def _sc_gather_fn():
  return pl.kernel(
    _sc_body,
    out_type=jax.ShapeDtypeStruct((B,), jnp.float32),
    mesh=plsc.VectorSubcoreMesh(core_axis_name="c", subcore_axis_name="s",
                                num_cores=NC, num_subcores=NS),
    compiler_params=pltpu.CompilerParams(needs_layout_passes=False),
    scratch_types=[
        pltpu.VMEM((N_PER_W,), jnp.int32),
        pltpu.VMEM((N_PER_W * DEG,), jnp.int32),
        pltpu.VMEM((2, 32, D), jnp.float32),
        pltpu.VMEM((NBUF, DEG, D), jnp.float32),
        pltpu.VMEM((D,), jnp.float32),
        pltpu.VMEM((E_PER_W, L), jnp.float32),
        pltpu.VMEM((E_PER_W,), jnp.float32),
        pltpu.VMEM_SHARED((N_NODES, D), jnp.float32),
        pltpu.SemaphoreType.DMA,
        pltpu.SemaphoreType.DMA,
        pltpu.SemaphoreType.DMA,
        pltpu.SemaphoreType.DMA,
    ],
  )


def kernel(edges, edge_pairs, neigh, features, W_enc, weight):
    w1 = W_enc[:, :D]
    w2 = W_enc[:, D:]
    h1, h2 = _tc_encode(features, w1, w2)
    node_list = jnp.take(edge_pairs, edges, axis=0).reshape(-1)
    neigh_flat = jnp.take(neigh, node_list, axis=0).reshape(-1)
    scores = _sc_gather_fn()(node_list, neigh_flat, h1, h2,
                             weight.reshape(D))
    return scores.reshape(B, 1)


# speedup vs baseline: 2.0443x; 1.2385x over previous
import functools

import jax
import jax.numpy as jnp
from jax import lax
from jax.experimental import pallas as pl
from jax.experimental.pallas import tpu as pltpu
from jax.experimental.pallas import tpu_sc as plsc

N_NODES = 10000
N_EDGES = 320000
D = 128
DEG = 32
B = 4096

NC = 2
NS = 16
NW = NC * NS
E_PER_W = B // NW
N_PER_W = 2 * E_PER_W
L = 16
VPD = D // L
NBUF = 4

_ROWS_BLK = 1000


def _tc_body(f_ref, w1_ref, w2_ref, h1_ref, h2_ref):
    f = f_ref[...]
    dn = (((1,), (1,)), ((), ()))
    h1_ref[...] = lax.dot_general(f, w1_ref[...], dn,
                                  preferred_element_type=jnp.float32)
    h2 = lax.dot_general(f, w2_ref[...], dn,
                         preferred_element_type=jnp.float32)
    h2_ref[...] = h2 * (1.0 / DEG)


def _tc_encode(features, w_enc):
    grid = (N_NODES // _ROWS_BLK,)
    return pl.pallas_call(
        _tc_body,
        grid=grid,
        in_specs=[
            pl.BlockSpec((_ROWS_BLK, D), lambda i: (i, 0)),
            pl.BlockSpec((D, D), lambda i: (0, 0)),
            pl.BlockSpec((D, D), lambda i: (0, 1)),
        ],
        out_specs=[
            pl.BlockSpec((_ROWS_BLK, D), lambda i: (i, 0)),
            pl.BlockSpec((_ROWS_BLK, D), lambda i: (i, 0)),
        ],
        out_shape=[
            jax.ShapeDtypeStruct((N_NODES, D), jnp.float32),
            jax.ShapeDtypeStruct((N_NODES, D), jnp.float32),
        ],
    )(features, w_enc, w_enc)


def _sc_body(node_hbm, nidx_hbm, h1_hbm, h2_hbm, w_hbm, out_hbm,
             node_v, neigh_v, sfb, bufs, wv_v, sc_v, out_v, h2_sh,
             semf0, semf1, sem0, sem1, sem2, sem3):
    sems = (sem0, sem1, sem2, sem3)
    semf = (semf0, semf1)
    sid = lax.axis_index("s")
    wid = sid * NC + lax.axis_index("c")
    base_e = wid * E_PER_W
    base_n = wid * N_PER_W
    NCH = N_PER_W // 32

    pltpu.sync_copy(node_hbm.at[pl.ds(base_n, N_PER_W)], node_v)
    pltpu.sync_copy(nidx_hbm.at[pl.ds(base_n * DEG, N_PER_W * DEG)], neigh_v)
    pltpu.sync_copy(w_hbm, wv_v)

    stage_base = pl.multiple_of(sid * 624, 8)
    pltpu.sync_copy(h2_hbm.at[pl.ds(stage_base, 624)],
                    h2_sh.at[pl.ds(stage_base, 624)])

    @pl.when(sid == 0)
    def _tail():
        pltpu.sync_copy(h2_hbm.at[pl.ds(624 * NS, N_NODES - 624 * NS)],
                        h2_sh.at[pl.ds(624 * NS, N_NODES - 624 * NS)])
    plsc.subcore_barrier()

    for cc in range(2):
        pltpu.async_copy(h1_hbm.at[node_v.at[pl.ds(cc * 32, 32)]],
                         sfb.at[cc], semf[cc])
    for b in range(NBUF):
        pltpu.async_copy(h2_sh.at[neigh_v.at[pl.ds(b * DEG, DEG)]],
                         bufs.at[b], sems[b])

    w_regs = [wv_v[pl.ds(v * L, L)] for v in range(VPD)]

    def _outer(i, _):
        for cc in range(2):
            c = i * 2 + cc
            pltpu.make_async_copy(h1_hbm.at[node_v.at[pl.ds(c * 32, 32)]],
                                  sfb.at[cc], semf[cc]).wait()

            def _inner(k, _2):
                rr = [None] * (NBUF // 2)
                for b in range(NBUF):
                    nl = k * NBUF + b
                    n = c * 32 + nl
                    pltpu.make_async_copy(
                        h2_sh.at[neigh_v.at[pl.ds(n * DEG, DEG)]],
                        bufs.at[b], sems[b]).wait()
                    def _accum(jj, a):
                        out = list(a)
                        for dj in range(4):
                            for v in range(VPD):
                                out[v] = out[v] + bufs[b, jj * 4 + dj,
                                                       pl.ds(v * L, L)]
                        return tuple(out)

                    acc = lax.fori_loop(
                        0, DEG // 4, _accum,
                        tuple(jnp.zeros((L,), jnp.float32)
                              for _ in range(VPD)))
                    nn = n + NBUF

                    @pl.when(nn < N_PER_W)
                    def _issue():
                        pltpu.async_copy(
                            h2_sh.at[neigh_v.at[pl.ds(nn * DEG, DEG)]],
                            bufs.at[b], sems[b])

                    r = None
                    for v in range(VPD):
                        z = jnp.maximum(acc[v] + sfb[cc, nl, pl.ds(v * L, L)],
                                        0.0)
                        t = z * w_regs[v]
                        r = t if r is None else r + t
                    e_half = b & 1
                    if e_half == 0:
                        rr[b // 2] = r
                    else:
                        edge = c * 16 + k * (NBUF // 2) + b // 2
                        sc_v[edge, pl.ds(0, L)] = rr[b // 2] + r
                return _2

            lax.fori_loop(0, 32 // NBUF, _inner, None)

            nc2 = c + 2

            @pl.when(nc2 < NCH)
            def _issue_sf():
                pltpu.async_copy(h1_hbm.at[node_v.at[pl.ds(nc2 * 32, 32)]],
                                 sfb.at[cc], semf[cc])
        return _

    lax.fori_loop(0, NCH // 2, _outer, None)

    for blk in range(E_PER_W // L):
        p = lax.iota(jnp.int32, L) + jnp.int32(blk * L)
        acc = None
        for cidx in range(L):
            cv = jnp.full((L,), cidx, jnp.int32)
            t = plsc.load_gather(sc_v, [p, cv])
            acc = t if acc is None else acc + t
        out_v[pl.ds(blk * L, L)] = acc * 0.5

    pltpu.sync_copy(out_v, out_hbm.at[pl.ds(base_e, E_PER_W)])


@functools.cache
def _sc_gather_fn():
  return pl.kernel(
    _sc_body,
    out_type=jax.ShapeDtypeStruct((B,), jnp.float32),
    mesh=plsc.VectorSubcoreMesh(core_axis_name="c", subcore_axis_name="s",
                                num_cores=NC, num_subcores=NS),
    compiler_params=pltpu.CompilerParams(needs_layout_passes=False),
    scratch_types=[
        pltpu.VMEM((N_PER_W,), jnp.int32),
        pltpu.VMEM((N_PER_W * DEG,), jnp.int32),
        pltpu.VMEM((2, 32, D), jnp.float32),
        pltpu.VMEM((NBUF, DEG, D), jnp.float32),
        pltpu.VMEM((D,), jnp.float32),
        pltpu.VMEM((E_PER_W, L), jnp.float32),
        pltpu.VMEM((E_PER_W,), jnp.float32),
        pltpu.VMEM_SHARED((N_NODES, D), jnp.float32),
        pltpu.SemaphoreType.DMA,
        pltpu.SemaphoreType.DMA,
        pltpu.SemaphoreType.DMA,
        pltpu.SemaphoreType.DMA,
        pltpu.SemaphoreType.DMA,
        pltpu.SemaphoreType.DMA,
    ],
  )


def kernel(edges, edge_pairs, neigh, features, W_enc, weight):
    h1, h2 = _tc_encode(features, W_enc)
    node_list = edge_pairs.at[edges].get(
        mode='promise_in_bounds').reshape(-1)
    neigh_flat = neigh.at[node_list].get(
        mode='promise_in_bounds').reshape(-1)
    scores = _sc_gather_fn()(node_list, neigh_flat, h1, h2,
                             weight.reshape(D))
    return scores.reshape(B, 1)


# speedup vs baseline: 2.1029x; 1.0286x over previous
import functools

import jax
import jax.numpy as jnp
from jax import lax
from jax.experimental import pallas as pl
from jax.experimental.pallas import tpu as pltpu
from jax.experimental.pallas import tpu_sc as plsc

N_NODES = 10000
N_EDGES = 320000
D = 128
DEG = 32
B = 4096

NC = 2
NS = 16
NW = NC * NS
E_PER_W = B // NW
N_PER_W = 2 * E_PER_W
L = 16
VPD = D // L
NBUF = 4

_ROWS_BLK = 1000


def _tc_body(f_ref, w1_ref, w2_ref, h1_ref, h2_ref):
    f = f_ref[...]
    dn = (((1,), (1,)), ((), ()))
    h1_ref[...] = lax.dot_general(f, w1_ref[...], dn,
                                  preferred_element_type=jnp.float32)
    h2 = lax.dot_general(f, w2_ref[...], dn,
                         preferred_element_type=jnp.float32)
    h2_ref[...] = h2 * (1.0 / DEG)


def _tc_encode(features, w_enc):
    grid = (N_NODES // _ROWS_BLK,)
    return pl.pallas_call(
        _tc_body,
        grid=grid,
        in_specs=[
            pl.BlockSpec((_ROWS_BLK, D), lambda i: (i, 0)),
            pl.BlockSpec((D, D), lambda i: (0, 0)),
            pl.BlockSpec((D, D), lambda i: (0, 1)),
        ],
        out_specs=[
            pl.BlockSpec((_ROWS_BLK, D), lambda i: (i, 0)),
            pl.BlockSpec((_ROWS_BLK, D), lambda i: (i, 0)),
        ],
        out_shape=[
            jax.ShapeDtypeStruct((N_NODES, D), jnp.float32),
            jax.ShapeDtypeStruct((N_NODES, D), jnp.float32),
        ],
    )(features, w_enc, w_enc)


def _sc_body(node_hbm, nidx_hbm, h1_hbm, h2_hbm, w_hbm, out_hbm,
             node_v, neigh_v, sfb, bufs, wv_v, sc_v, out_v, h2_sh,
             semf0, semf1, sem_st, sem0, sem1, sem2, sem3):
    sems = (sem0, sem1, sem2, sem3)
    semf = (semf0, semf1)
    sid = lax.axis_index("s")
    wid = sid * NC + lax.axis_index("c")
    base_e = wid * E_PER_W
    base_n = wid * N_PER_W
    NCH = N_PER_W // 32

    stage_base = pl.multiple_of(sid * 624, 8)
    stage_cp = pltpu.async_copy(h2_hbm.at[pl.ds(stage_base, 624)],
                                h2_sh.at[pl.ds(stage_base, 624)], sem_st)

    pltpu.sync_copy(node_hbm.at[pl.ds(base_n, N_PER_W)], node_v)
    pltpu.sync_copy(nidx_hbm.at[pl.ds(base_n * DEG, N_PER_W * DEG)], neigh_v)
    pltpu.sync_copy(w_hbm, wv_v)

    for cc in range(2):
        pltpu.async_copy(h1_hbm.at[node_v.at[pl.ds(cc * 32, 32)]],
                         sfb.at[cc], semf[cc])

    @pl.when(sid == 0)
    def _tail():
        pltpu.sync_copy(h2_hbm.at[pl.ds(624 * NS, N_NODES - 624 * NS)],
                        h2_sh.at[pl.ds(624 * NS, N_NODES - 624 * NS)])
    stage_cp.wait()
    plsc.subcore_barrier()

    for b in range(NBUF):
        pltpu.async_copy(h2_sh.at[neigh_v.at[pl.ds(b * DEG, DEG)]],
                         bufs.at[b], sems[b])

    w_regs = [wv_v[pl.ds(v * L, L)] for v in range(VPD)]

    def _outer(i, _):
        for cc in range(2):
            c = i * 2 + cc
            pltpu.make_async_copy(h1_hbm.at[node_v.at[pl.ds(c * 32, 32)]],
                                  sfb.at[cc], semf[cc]).wait()

            def _inner(k, _2):
                rr = [None] * (NBUF // 2)
                for b in range(NBUF):
                    nl = k * NBUF + b
                    n = c * 32 + nl
                    pltpu.make_async_copy(
                        h2_sh.at[neigh_v.at[pl.ds(n * DEG, DEG)]],
                        bufs.at[b], sems[b]).wait()
                    def _accum(jj, a):
                        out = list(a)
                        for dj in range(4):
                            for v in range(VPD):
                                out[v] = out[v] + bufs[b, jj * 4 + dj,
                                                       pl.ds(v * L, L)]
                        return tuple(out)

                    acc = lax.fori_loop(
                        0, DEG // 4, _accum,
                        tuple(jnp.zeros((L,), jnp.float32)
                              for _ in range(VPD)))
                    nn = n + NBUF

                    @pl.when(nn < N_PER_W)
                    def _issue():
                        pltpu.async_copy(
                            h2_sh.at[neigh_v.at[pl.ds(nn * DEG, DEG)]],
                            bufs.at[b], sems[b])

                    r = None
                    for v in range(VPD):
                        z = jnp.maximum(acc[v] + sfb[cc, nl, pl.ds(v * L, L)],
                                        0.0)
                        t = z * w_regs[v]
                        r = t if r is None else r + t
                    e_half = b & 1
                    if e_half == 0:
                        rr[b // 2] = r
                    else:
                        edge = c * 16 + k * (NBUF // 2) + b // 2
                        sc_v[edge, pl.ds(0, L)] = rr[b // 2] + r
                return _2

            lax.fori_loop(0, 32 // NBUF, _inner, None)

            nc2 = c + 2

            @pl.when(nc2 < NCH)
            def _issue_sf():
                pltpu.async_copy(h1_hbm.at[node_v.at[pl.ds(nc2 * 32, 32)]],
                                 sfb.at[cc], semf[cc])
        return _

    lax.fori_loop(0, NCH // 2, _outer, None)

    for blk in range(E_PER_W // L):
        p = lax.iota(jnp.int32, L) + jnp.int32(blk * L)
        acc = None
        for cidx in range(L):
            cv = jnp.full((L,), cidx, jnp.int32)
            t = plsc.load_gather(sc_v, [p, cv])
            acc = t if acc is None else acc + t
        out_v[pl.ds(blk * L, L)] = acc * 0.5

    pltpu.sync_copy(out_v, out_hbm.at[pl.ds(base_e, E_PER_W)])


@functools.cache
def _sc_gather_fn():
  return pl.kernel(
    _sc_body,
    out_type=jax.ShapeDtypeStruct((B,), jnp.float32),
    mesh=plsc.VectorSubcoreMesh(core_axis_name="c", subcore_axis_name="s",
                                num_cores=NC, num_subcores=NS),
    compiler_params=pltpu.CompilerParams(needs_layout_passes=False),
    scratch_types=[
        pltpu.VMEM((N_PER_W,), jnp.int32),
        pltpu.VMEM((N_PER_W * DEG,), jnp.int32),
        pltpu.VMEM((2, 32, D), jnp.float32),
        pltpu.VMEM((NBUF, DEG, D), jnp.float32),
        pltpu.VMEM((D,), jnp.float32),
        pltpu.VMEM((E_PER_W, L), jnp.float32),
        pltpu.VMEM((E_PER_W,), jnp.float32),
        pltpu.VMEM_SHARED((N_NODES, D), jnp.float32),
        pltpu.SemaphoreType.DMA,
        pltpu.SemaphoreType.DMA,
        pltpu.SemaphoreType.DMA,
        pltpu.SemaphoreType.DMA,
        pltpu.SemaphoreType.DMA,
        pltpu.SemaphoreType.DMA,
        pltpu.SemaphoreType.DMA,
    ],
  )


def kernel(edges, edge_pairs, neigh, features, W_enc, weight):
    h1, h2 = _tc_encode(features, W_enc)
    node_list = edge_pairs.at[edges].get(
        mode='promise_in_bounds').reshape(-1)
    neigh_flat = neigh.at[node_list].get(
        mode='promise_in_bounds').reshape(-1)
    scores = _sc_gather_fn()(node_list, neigh_flat, h1, h2,
                             weight.reshape(D))
    return scores.reshape(B, 1)
